# baseline (device time: 28155 ns/iter reference)
import jax
import jax.numpy as jnp
from jax import lax
from jax.experimental import pallas as pl
from jax.experimental.pallas import tpu as pltpu

N_DEV = 4
N_LAYERS = 3
B = 512
D = 256
M = B // N_DEV


def kernel(x, Win0, Wout0, Win1, Wout1, Win2, Wout2):
    def body(x_ref, win0, wout0, win1, wout1, win2, wout2,
             out_ref, part_ref, piece_ref, xin_ref, xout_ref,
             psend_sems, precv_sems, xsend_sems, xrecv_sems):
        my = lax.axis_index("i")
        right = lax.rem(my + 1, N_DEV)
        diag = lax.rem(my + 2, N_DEV)
        left = lax.rem(my + 3, N_DEV)

        barrier_sem = pltpu.get_barrier_semaphore()
        for d in range(1, N_DEV):
            pl.semaphore_signal(
                barrier_sem, inc=1,
                device_id=(lax.rem(my + d, N_DEV),),
                device_id_type=pl.DeviceIdType.MESH,
            )

        pending_sends = []

        def send_piece(k, t_rows, target, slot):
            rdma = pltpu.make_async_remote_copy(
                src_ref=part_ref.at[k].at[pl.ds(t_rows * M, M)],
                dst_ref=piece_ref.at[k, slot],
                send_sem=psend_sems.at[k, slot],
                recv_sem=precv_sems.at[k, slot],
                device_id=(target,),
                device_id_type=pl.DeviceIdType.MESH,
            )
            rdma.start()
            pending_sends.append(rdma)

        def wait_piece(k, slot):
            rdma = pltpu.make_async_remote_copy(
                src_ref=piece_ref.at[k, slot],
                dst_ref=piece_ref.at[k, slot],
                send_sem=psend_sems.at[k, slot],
                recv_sem=precv_sems.at[k, slot],
                device_id=(my,),
                device_id_type=pl.DeviceIdType.MESH,
            )
            rdma.wait_recv()
            return piece_ref[k, slot].astype(jnp.float32)

        def send_x(k, target, slot):
            rdma = pltpu.make_async_remote_copy(
                src_ref=xout_ref.at[k],
                dst_ref=xin_ref.at[k, slot],
                send_sem=xsend_sems.at[k, slot],
                recv_sem=xrecv_sems.at[k, slot],
                device_id=(target,),
                device_id_type=pl.DeviceIdType.MESH,
            )
            rdma.start()
            pending_sends.append(rdma)

        def wait_x(k, slot):
            rdma = pltpu.make_async_remote_copy(
                src_ref=xin_ref.at[k, slot],
                dst_ref=xin_ref.at[k, slot],
                send_sem=xsend_sems.at[k, slot],
                recv_sem=xrecv_sems.at[k, slot],
                device_id=(my,),
                device_id_type=pl.DeviceIdType.MESH,
            )
            rdma.wait_recv()
            return xin_ref[k, slot]

        wins = [win0, win1, win2]
        wouts = [wout0, wout1, wout2]
        red_a = None
        red_b = None

        for k in range(N_LAYERS):
            wi = wins[k][...].astype(jnp.bfloat16)
            wo = wouts[k][...].astype(jnp.bfloat16)
            last = k == N_LAYERS - 1

            def piece(xt):
                hh = jnp.dot(xt, wi, preferred_element_type=jnp.float32)
                hh = jnp.maximum(hh, 0.0).astype(jnp.bfloat16)
                return jnp.dot(hh, wo, preferred_element_type=jnp.float32)

            if k == 0:
                xa = x_ref[pl.ds(my * M, M), :].astype(jnp.bfloat16)
                xb = x_ref[pl.ds(diag * M, M), :].astype(jnp.bfloat16)
            else:
                xa = red_a.astype(jnp.bfloat16)
                xb = red_b.astype(jnp.bfloat16)
            part_ref[k, pl.ds(diag * M, M), :] = piece(xb).astype(jnp.bfloat16)
            if k == 0:
                pl.semaphore_wait(barrier_sem, N_DEV - 1)
            send_piece(k, diag, diag, 0)
            part_ref[k, pl.ds(my * M, M), :] = piece(xa).astype(jnp.bfloat16)
            if not last:
                send_piece(k, my, diag, 1)

            if k == 0:
                xr = x_ref[pl.ds(right * M, M), :].astype(jnp.bfloat16)
            else:
                xr = wait_x(k - 1, 0)
            part_ref[k, pl.ds(right * M, M), :] = piece(xr).astype(jnp.bfloat16)
            send_piece(k, right, right, 4)
            if not last:
                send_piece(k, right, left, 3)

            if k == 0:
                xl = x_ref[pl.ds(left * M, M), :].astype(jnp.bfloat16)
            else:
                xl = wait_x(k - 1, 1)
            part_ref[k, pl.ds(left * M, M), :] = piece(xl).astype(jnp.bfloat16)
            send_piece(k, left, left, 2)
            if not last:
                send_piece(k, left, right, 5)

            red_a = part_ref[k, pl.ds(my * M, M), :].astype(jnp.float32)
            for slot in (0, 2, 4):
                red_a = red_a + wait_piece(k, slot)

            if last:
                out_ref[...] = red_a
            else:
                xout_ref[k, :, :] = red_a.astype(jnp.bfloat16)
                send_x(k, right, 1)
                send_x(k, left, 0)
                red_b = part_ref[k, pl.ds(diag * M, M), :].astype(jnp.float32)
                for slot in (1, 3, 5):
                    red_b = red_b + wait_piece(k, slot)

        for rdma in pending_sends:
            rdma.wait_send()

    return pl.pallas_call(
        body,
        out_shape=jax.ShapeDtypeStruct((M, D), jnp.float32),
        in_specs=[pl.BlockSpec(memory_space=pltpu.VMEM)] * 7,
        out_specs=pl.BlockSpec(memory_space=pltpu.VMEM),
        scratch_shapes=[
            pltpu.VMEM((N_LAYERS, B, D), jnp.bfloat16),
            pltpu.VMEM((N_LAYERS, 6, M, D), jnp.bfloat16),
            pltpu.VMEM((N_LAYERS - 1, 2, M, D), jnp.bfloat16),
            pltpu.VMEM((N_LAYERS - 1, M, D), jnp.bfloat16),
            pltpu.SemaphoreType.DMA((N_LAYERS, 6)),
            pltpu.SemaphoreType.DMA((N_LAYERS, 6)),
            pltpu.SemaphoreType.DMA((N_LAYERS - 1, 2)),
            pltpu.SemaphoreType.DMA((N_LAYERS - 1, 2)),
        ],
        compiler_params=pltpu.CompilerParams(collective_id=0),
    )(x, Win0, Wout0, Win1, Wout1, Win2, Wout2)
